# Initial kernel scaffold; baseline (speedup 1.0000x reference)
import jax
import jax.numpy as jnp
from jax.experimental import pallas as pl


def kernel(x, edge_index, W, b):
    raise NotImplementedError("write your pallas kernel here")


# speedup vs baseline: 3.6445x; 3.6445x over previous
import functools

import jax
import jax.numpy as jnp
from jax import lax
from jax.experimental import pallas as pl
from jax.experimental.pallas import tpu as pltpu
from jax.experimental.pallas import tpu_sc as plsc

N = 10000
D = 128
E = 320000
DP = 144
NP = 10112
NC = 2
NS = 16
NW = NC * NS
CH = 128
SUP = 8
NSUP = 10
NCHUNK = SUP * NSUP
EPT = CH * NCHUNK
RPT = NP // NS
RB = 1000


def _tf_body(x_ref, w_ref, b_ref, o_ref):
    xb = x_ref[...]
    sq = jnp.sum(xb * xb, axis=1, keepdims=True)
    norm = jnp.sqrt(sq)
    nc = jnp.clip(norm, 1e-10, 1.0 - 1e-5)
    atanh = 0.5 * jnp.log((1.0 + nc) / (1.0 - nc))
    t = xb * (atanh / jnp.maximum(norm, 1e-10))
    y = lax.dot_general(t, w_ref[...], (((1,), (1,)), ((), ())),
                        preferred_element_type=jnp.float32)
    o_ref[:, :D] = y + b_ref[...]
    col = lax.broadcasted_iota(jnp.int32, (RB, DP - D), 1)
    o_ref[:, D:] = jnp.where(col == 0, 1.0, 0.0)


_transform = pl.pallas_call(
    _tf_body,
    grid=(N // RB,),
    in_specs=[
        pl.BlockSpec((RB, D), lambda i: (i, 0)),
        pl.BlockSpec((D, D), lambda i: (0, 0)),
        pl.BlockSpec((1, D), lambda i: (0, 0)),
    ],
    out_specs=pl.BlockSpec((RB, DP), lambda i: (i, 0)),
    out_shape=jax.ShapeDtypeStruct((N, DP), jnp.float32),
)


def _sc_body(tr_hbm, src_hbm, dst_hbm, z_hbm, acc_hbm,
             acc_sh, src_v, dst_v, rows, sems):
    c = lax.axis_index("c")
    s = lax.axis_index("s")
    w = c * NS + s
    pltpu.sync_copy(z_hbm.at[pl.ds(s * RPT, RPT)], acc_sh.at[pl.ds(s * RPT, RPT)])
    plsc.subcore_barrier()

    @pl.loop(0, NSUP)
    def _(sj):
        pltpu.sync_copy(src_hbm.at[w, pl.ds(sj * SUP, SUP)], src_v)
        pltpu.sync_copy(dst_hbm.at[w, pl.ds(sj * SUP, SUP)], dst_v)
        pltpu.make_async_copy(tr_hbm.at[src_v.at[0]], rows[0], sems[0]).start()
        for k in range(SUP):
            b = k % 2
            pltpu.make_async_copy(tr_hbm.at[src_v.at[k]], rows[b], sems[b]).wait()
            if k + 1 < SUP:
                nb = (k + 1) % 2
                pltpu.make_async_copy(
                    tr_hbm.at[src_v.at[k + 1]], rows[nb], sems[nb]).start()
            pltpu.sync_copy(rows[b], acc_sh.at[dst_v.at[k]], add=True)

    plsc.subcore_barrier()
    pltpu.sync_copy(acc_sh.at[pl.ds(s * RPT, RPT)],
                    acc_hbm.at[c, pl.ds(s * RPT, RPT)])


@functools.lru_cache(maxsize=1)
def _make_scatter():
    return pl.kernel(
        _sc_body,
        out_type=jax.ShapeDtypeStruct((NC, NP, DP), jnp.float32),
        mesh=plsc.VectorSubcoreMesh(core_axis_name="c", subcore_axis_name="s",
                                    num_cores=NC, num_subcores=NS),
        compiler_params=pltpu.CompilerParams(use_tc_tiling_on_sc=False),
        scratch_types=[
            pltpu.VMEM_SHARED((NP, DP), jnp.float32),
            pltpu.VMEM((SUP, CH), jnp.int32),
            pltpu.VMEM((SUP, CH), jnp.int32),
            [pltpu.VMEM((CH, DP), jnp.float32),
             pltpu.VMEM((CH, DP), jnp.float32)],
            [pltpu.SemaphoreType.DMA, pltpu.SemaphoreType.DMA],
        ],
    )


def _fin_body(a_ref, o_ref):
    a = a_ref[0] + a_ref[1]
    cnt = a[:, D:D + 1]
    v = a[:, :D] / jnp.maximum(cnt, 1.0)
    norm = jnp.sqrt(jnp.sum(v * v, axis=1, keepdims=True))
    o_ref[...] = jnp.tanh(norm) * v / jnp.maximum(norm, 1e-10)


_finish = pl.pallas_call(
    _fin_body,
    grid=(N // RB,),
    in_specs=[pl.BlockSpec((NC, RB, DP), lambda i: (0, i, 0))],
    out_specs=pl.BlockSpec((RB, D), lambda i: (i, 0)),
    out_shape=jax.ShapeDtypeStruct((N, D), jnp.float32),
)


def kernel(x, edge_index, W, b):
    src = edge_index[0].astype(jnp.int32)
    dst = edge_index[1].astype(jnp.int32)
    pad = NW * EPT - E
    src_p = jnp.concatenate([src, jnp.zeros((pad,), jnp.int32)]).reshape(NW, NCHUNK, CH)
    dst_p = jnp.concatenate([dst, jnp.full((pad,), NP - 1, jnp.int32)]).reshape(NW, NCHUNK, CH)
    zeros = jnp.zeros((NP, DP), jnp.float32)
    tr = _transform(x, W, b.reshape(1, D))
    acc = _make_scatter()(tr, src_p, dst_p, zeros)
    return _finish(acc)
